# Initial kernel scaffold; baseline (speedup 1.0000x reference)
import jax
import jax.numpy as jnp
from jax.experimental import pallas as pl


def kernel(x, embed):
    raise NotImplementedError("write your pallas kernel here")


# speedup vs baseline: 1.6582x; 1.6582x over previous
import functools

import jax
import jax.numpy as jnp
from jax import lax
from jax.experimental import pallas as pl
from jax.experimental.pallas import tpu as pltpu


def _vq_body(x_ref, embed_ref, q_ref, ind_ref, diff_ref):
    f = x_ref[...]
    c = embed_ref[...]
    ab = lax.dot_general(2.0 * f, c, (((1,), (1,)), ((), ())),
                         preferred_element_type=jnp.float32)
    f2 = jnp.sum(f * f, axis=1, keepdims=True)
    c2 = jnp.sum(c * c, axis=1)[None, :]
    dist = (f2 - ab) + c2
    m = jnp.min(dist, axis=1, keepdims=True)
    kidx = lax.broadcasted_iota(jnp.int32, dist.shape, 1)
    ind = jnp.min(jnp.where(dist <= m, kidx, dist.shape[1]), axis=1)
    onehot = (kidx == ind[:, None]).astype(jnp.float32)
    q = lax.dot_general(onehot, c, (((1,), (0,)), ((), ())),
                        preferred_element_type=jnp.float32)
    q_ref[...] = q
    ind_ref[...] = ind
    diff_ref[...] = q - f


@functools.partial(jax.jit, static_argnames=())
def kernel(x, embed):
    n0, n1, d = x.shape
    k = embed.shape[0]
    flat = x.reshape(-1, d)
    n = flat.shape[0]
    tn = 2048
    grid = (n // tn,)
    q, ind, diff = pl.pallas_call(
        _vq_body,
        grid=grid,
        in_specs=[
            pl.BlockSpec((tn, d), lambda i: (i, 0)),
            pl.BlockSpec((k, d), lambda i: (0, 0)),
        ],
        out_specs=[
            pl.BlockSpec((tn, d), lambda i: (i, 0)),
            pl.BlockSpec((tn,), lambda i: (i,)),
            pl.BlockSpec((tn, d), lambda i: (i, 0)),
        ],
        out_shape=[
            jax.ShapeDtypeStruct((n, d), jnp.float32),
            jax.ShapeDtypeStruct((n,), jnp.int32),
            jax.ShapeDtypeStruct((n, d), jnp.float32),
        ],
    )(flat, embed)
    return (q, ind, diff)
